# baseline (speedup 1.0000x reference)
import functools

import jax
import jax.numpy as jnp
from jax.experimental import pallas as pl
from jax.experimental.pallas import tpu as pltpu


def _matmul_body(x_ref, w_ref, b_ref, o_ref):
    o_ref[...] = (
        jnp.dot(x_ref[...], w_ref[...], preferred_element_type=jnp.float32)
        + b_ref[...]
    )


def _pallas_matmul(x, w_t, b, blk=512):
    n, c = x.shape
    co = w_t.shape[1]
    return pl.pallas_call(
        _matmul_body,
        grid=(n // blk,),
        in_specs=[
            pl.BlockSpec((blk, c), lambda i: (i, 0)),
            pl.BlockSpec((c, co), lambda i: (0, 0)),
            pl.BlockSpec((1, co), lambda i: (0, 0)),
        ],
        out_specs=pl.BlockSpec((blk, co), lambda i: (i, 0)),
        out_shape=jax.ShapeDtypeStruct((n, co), jnp.float32),
    )(x, w_t, b.reshape(1, co))


def _bilinear_sample(value, pts):
    bs, C, H, Wd = value.shape
    x = pts[..., 0] * Wd - 0.5
    y = pts[..., 1] * H - 0.5
    x0 = jnp.floor(x)
    y0 = jnp.floor(y)
    x1 = x0 + 1.0
    y1 = y0 + 1.0
    wx1 = x - x0
    wx0 = 1.0 - wx1
    wy1 = y - y0
    wy0 = 1.0 - wy1
    flat = value.reshape(bs, C, H * Wd)

    def gather(xi, yi):
        valid = (xi >= 0) & (xi <= Wd - 1) & (yi >= 0) & (yi <= H - 1)
        xc = jnp.clip(xi, 0, Wd - 1).astype(jnp.int32)
        yc = jnp.clip(yi, 0, H - 1).astype(jnp.int32)
        idx = yc * Wd + xc
        g = jnp.take_along_axis(flat, idx[:, None, :], axis=2)
        return g * valid[:, None, :].astype(value.dtype)

    out = (gather(x0, y0) * (wx0 * wy0)[:, None, :]
           + gather(x1, y0) * (wx1 * wy0)[:, None, :]
           + gather(x0, y1) * (wx0 * wy1)[:, None, :]
           + gather(x1, y1) * (wx1 * wy1)[:, None, :])
    return out.transpose(0, 2, 1)


@jax.jit
def kernel(feats, anchor_points, W_value, b_value, W_weights, b_weights,
           W_offset, b_offset, W_out, b_out):
    bs, C, H, Wd = feats.shape
    G, P = 8, 9
    Cg = C // G
    HW = H * Wd
    x = feats.reshape(bs, C, HW).transpose(0, 2, 1).reshape(bs * HW, C)

    v = _pallas_matmul(x, W_value.T, b_value)
    wraw = x @ W_weights.T + b_weights
    off = x @ W_offset.T + b_offset

    w = wraw.reshape(bs, HW, P, G)
    w = jax.nn.softmax(w, axis=2)

    kp = (anchor_points[:, :, None, :]
          + off.reshape(bs, HW, P, 2)).reshape(bs, HW * P, 2)
    feats_value = v.reshape(bs, HW, C).transpose(0, 2, 1).reshape(bs, C, H, Wd)
    sampled = _bilinear_sample(feats_value, kp)
    weighted = (sampled.reshape(bs, HW * P, G, Cg)
                * w.reshape(bs, HW * P, G, 1))
    out = weighted.reshape(bs, HW, P, C).sum(axis=2)

    out = _pallas_matmul(out.reshape(bs * HW, C), W_out.T, b_out)
    out = out.reshape(bs, HW, C).transpose(0, 2, 1).reshape(bs, C, H, Wd)
    return out


# speedup vs baseline: 10.8185x; 10.8185x over previous
import functools

import jax
import jax.numpy as jnp
import numpy as np
from jax import lax
from jax.experimental import pallas as pl
from jax.experimental.pallas import tpu as pltpu
from jax.experimental.pallas import tpu_sc as plsc

C = 384
G = 8
P = 9
NCORN = 4
BS = 4
H = 32
W = 32
HW = H * W
ROWS = BS * HW
LANE = 16
NSLAB = C // LANE
HALF = HW // 2
NTASK = BS * NSLAB * 2
NWORK = 32
TASKS_PER_W = NTASK // NWORK

_ROWBLK = 512


def _prep_body(x_ref, wv_ref, bv_ref, wsm_ref, bsm_ref, seg_ref,
               wx_ref, wy_ref, bxy_ref, axy_ref,
               v_ref, idx_ref, bw_ref, sm_ref):
    x = x_ref[...]
    v_ref[...] = (jnp.dot(x, wv_ref[...], preferred_element_type=jnp.float32)
                  + bv_ref[...])
    e = jnp.exp(jnp.dot(x, wsm_ref[...], preferred_element_type=jnp.float32)
                + bsm_ref[...])
    z = jnp.dot(e, seg_ref[...], preferred_element_type=jnp.float32)
    sm_ref[...] = e / z
    xp = (jnp.dot(x, wx_ref[...], preferred_element_type=jnp.float32)
          + bxy_ref[0:1, :] + axy_ref[:, 0:1])
    yp = (jnp.dot(x, wy_ref[...], preferred_element_type=jnp.float32)
          + bxy_ref[1:2, :] + axy_ref[:, 1:2])
    lane = lax.broadcasted_iota(jnp.int32, xp.shape, 1)
    cidx = lane >> 4
    is_x1 = (cidx & 1) == 1
    is_y1 = cidx >= 2
    one = jnp.float32(1.0)
    xf = jnp.floor(xp)
    dx = xp - xf
    xc = xf + jnp.where(is_x1, one, 0.0)
    wxc = jnp.where(is_x1, dx, one - dx)
    yf = jnp.floor(yp)
    dy = yp - yf
    yc = yf + jnp.where(is_y1, one, 0.0)
    wyc = jnp.where(is_y1, dy, one - dy)
    valid = ((xc >= 0) & (xc <= W - 1) & (yc >= 0) & (yc <= H - 1))
    bw = wxc * wyc * jnp.where(valid, one, 0.0)
    xi = jnp.clip(xc, 0, W - 1).astype(jnp.int32)
    yi = jnp.clip(yc, 0, H - 1).astype(jnp.int32)
    idx = yi * W + xi
    idx_ref[...] = idx[:, 0:64]
    bw_ref[...] = bw[:, 0:64]


def _prep(x, wv, bv, wsm, bsm, seg, wx, wy, bxy, axy):
    nblk = ROWS // _ROWBLK
    row_spec = lambda nc: pl.BlockSpec((_ROWBLK, nc), lambda i: (i, 0))
    full = lambda a: pl.BlockSpec(a.shape, lambda i: (0,) * a.ndim)
    return pl.pallas_call(
        _prep_body,
        grid=(nblk,),
        in_specs=[row_spec(C), full(wv), full(bv), full(wsm), full(bsm),
                  full(seg), full(wx), full(wy), full(bxy), row_spec(2)],
        out_specs=[row_spec(C), row_spec(64), row_spec(64), row_spec(128)],
        out_shape=[
            jax.ShapeDtypeStruct((ROWS, C), jnp.float32),
            jax.ShapeDtypeStruct((ROWS, 64), jnp.int32),
            jax.ShapeDtypeStruct((ROWS, 64), jnp.float32),
            jax.ShapeDtypeStruct((ROWS, 128), jnp.float32),
        ],
    )(x, wv, bv, wsm, bsm, seg, wx, wy, bxy, axy)


def _sc_body(v_hbm, idx_hbm, bw_hbm, sm_hbm, out_hbm,
             vslab, idx_v, bw_v, sm_v, out_v):
    wid = lax.axis_index("s") * 2 + lax.axis_index("c")
    iota16 = lax.iota(jnp.int32, LANE)
    for r in range(TASKS_PER_W):
        t = wid * TASKS_PER_W + r
        b = t // (NSLAB * 2)
        rem = t % (NSLAB * 2)
        slab = rem // 2
        half = rem % 2
        grp = slab // (NSLAB // G)
        row0 = b * HW + half * HALF
        col0 = slab * LANE
        pltpu.sync_copy(v_hbm.at[pl.ds(b * HW, HW), pl.ds(col0, LANE)],
                        vslab)
        pltpu.sync_copy(idx_hbm.at[pl.ds(row0, HALF)], idx_v)
        pltpu.sync_copy(bw_hbm.at[pl.ds(row0, HALF)], bw_v)
        pltpu.sync_copy(sm_hbm.at[pl.ds(row0, HALF), pl.ds(grp * LANE, LANE)],
                        sm_v)
        zeros16 = jnp.zeros((LANE,), jnp.int32)

        def loc(i, carry):
            smr = sm_v[i]
            idxr = [idx_v[i, pl.ds(c * LANE, LANE)] for c in range(NCORN)]
            bwr = [bw_v[i, pl.ds(c * LANE, LANE)] for c in range(NCORN)]
            acc = jnp.zeros((LANE,), jnp.float32)
            for p in range(P):
                smv = smr[p]
                ptmp = jnp.zeros((LANE,), jnp.float32)
                for c in range(NCORN):
                    rid = idxr[c][p]
                    bwv = bwr[c][p]
                    vals = plsc.load_gather(vslab, [rid + zeros16, iota16])
                    ptmp = ptmp + vals * bwv
                acc = acc + ptmp * smv
            out_v[i] = acc
            return carry

        lax.fori_loop(0, HALF, loc, 0)
        pltpu.sync_copy(out_v,
                        out_hbm.at[pl.ds(row0, HALF), pl.ds(col0, LANE)])


@functools.lru_cache(maxsize=1)
def _sc_aggregate_fn():
    return pl.kernel(
        _sc_body,
        out_type=jax.ShapeDtypeStruct((ROWS, C), jnp.float32),
        mesh=plsc.VectorSubcoreMesh(core_axis_name="c",
                                    subcore_axis_name="s"),
        compiler_params=pltpu.CompilerParams(use_tc_tiling_on_sc=False,
                                             needs_layout_passes=False),
        scratch_types=[
            pltpu.VMEM((HW, LANE), jnp.float32),
            pltpu.VMEM((HALF, 64), jnp.int32),
            pltpu.VMEM((HALF, 64), jnp.float32),
            pltpu.VMEM((HALF, LANE), jnp.float32),
            pltpu.VMEM((HALF, LANE), jnp.float32),
        ],
    )


def _out_body(x_ref, w_ref, b_ref, o_ref):
    o_ref[...] = (jnp.dot(x_ref[...], w_ref[...],
                          preferred_element_type=jnp.float32) + b_ref[...])


def _out_proj(x, w_t, b):
    nblk = ROWS // _ROWBLK
    return pl.pallas_call(
        _out_body,
        grid=(nblk,),
        in_specs=[pl.BlockSpec((_ROWBLK, C), lambda i: (i, 0)),
                  pl.BlockSpec((C, C), lambda i: (0, 0)),
                  pl.BlockSpec((1, C), lambda i: (0, 0))],
        out_specs=pl.BlockSpec((_ROWBLK, C), lambda i: (i, 0)),
        out_shape=jax.ShapeDtypeStruct((ROWS, C), jnp.float32),
    )(x, w_t, b.reshape(1, C))


def _lane_maps():
    sm_cols = np.zeros((G * P, 128), np.float32)
    for g in range(G):
        for p in range(P):
            sm_cols[p * G + g, g * LANE + p] = 1.0
    seg = np.zeros((128, 128), np.float32)
    for g in range(G):
        for p in range(P):
            for q in range(P):
                seg[g * LANE + p, g * LANE + q] = 1.0
    offx = np.zeros((2 * P, 128), np.float32)
    offy = np.zeros((2 * P, 128), np.float32)
    for c in range(NCORN):
        for p in range(P):
            offx[p * 2 + 0, c * LANE + p] = float(W)
            offy[p * 2 + 1, c * LANE + p] = float(H)
    return sm_cols, seg, offx, offy


_SM_COLS, _SEG, _OFFX, _OFFY = _lane_maps()


@jax.jit
def kernel(feats, anchor_points, W_value, b_value, W_weights, b_weights,
           W_offset, b_offset, W_out, b_out):
    x = feats.reshape(BS, C, HW).transpose(0, 2, 1).reshape(ROWS, C)

    sm_cols = jnp.asarray(_SM_COLS)
    seg = jnp.asarray(_SEG)
    wsm = W_weights.T @ sm_cols
    bsm = (b_weights @ sm_cols).reshape(1, 128)
    wx = W_offset.T @ jnp.asarray(_OFFX)
    wy = W_offset.T @ jnp.asarray(_OFFY)
    bxy = jnp.stack([b_offset @ jnp.asarray(_OFFX),
                     b_offset @ jnp.asarray(_OFFY)])
    axy = (anchor_points.reshape(ROWS, 2) * jnp.float32(W)
           - jnp.float32(0.5))

    v, idx, bw, sm = _prep(x, W_value.T, b_value.reshape(1, C),
                           wsm, bsm, seg, wx, wy, bxy, axy)
    out_pre = _sc_aggregate_fn()(v, idx, bw, sm)
    out = _out_proj(out_pre, W_out.T, b_out)
    return out.reshape(BS, HW, C).transpose(0, 2, 1).reshape(BS, C, H, W)


# speedup vs baseline: 12.4408x; 1.1500x over previous
import functools

import jax
import jax.numpy as jnp
import numpy as np
from jax import lax
from jax.experimental import pallas as pl
from jax.experimental.pallas import tpu as pltpu
from jax.experimental.pallas import tpu_sc as plsc

C = 384
G = 8
P = 9
NCORN = 4
BS = 4
H = 32
W = 32
HW = H * W
ROWS = BS * HW
LANE = 16
NSLAB = C // LANE
HALF = HW // 2
NTASK = BS * NSLAB * 2
NWORK = 32
TASKS_PER_W = NTASK // NWORK

_ROWBLK = 512


def _prep_body(x_ref, wv_ref, bv_ref, wsm_ref, bsm_ref, seg_ref,
               wx_ref, wy_ref, bxy_ref, axy_ref,
               v_ref, idx_ref, bw_ref, sm_ref):
    x = x_ref[...]
    v_ref[...] = (jnp.dot(x, wv_ref[...], preferred_element_type=jnp.float32)
                  + bv_ref[...])
    e = jnp.exp(jnp.dot(x, wsm_ref[...], preferred_element_type=jnp.float32)
                + bsm_ref[...])
    z = jnp.dot(e, seg_ref[...], preferred_element_type=jnp.float32)
    sm_ref[...] = e / z
    xp = (jnp.dot(x, wx_ref[...], preferred_element_type=jnp.float32)
          + bxy_ref[0:1, :] + axy_ref[:, 0:1])
    yp = (jnp.dot(x, wy_ref[...], preferred_element_type=jnp.float32)
          + bxy_ref[1:2, :] + axy_ref[:, 1:2])
    lane = lax.broadcasted_iota(jnp.int32, xp.shape, 1)
    cidx = lane >> 4
    is_x1 = (cidx & 1) == 1
    is_y1 = cidx >= 2
    one = jnp.float32(1.0)
    xf = jnp.floor(xp)
    dx = xp - xf
    xc = xf + jnp.where(is_x1, one, 0.0)
    wxc = jnp.where(is_x1, dx, one - dx)
    yf = jnp.floor(yp)
    dy = yp - yf
    yc = yf + jnp.where(is_y1, one, 0.0)
    wyc = jnp.where(is_y1, dy, one - dy)
    valid = ((xc >= 0) & (xc <= W - 1) & (yc >= 0) & (yc <= H - 1))
    bw = wxc * wyc * jnp.where(valid, one, 0.0)
    xi = jnp.clip(xc, 0, W - 1).astype(jnp.int32)
    yi = jnp.clip(yc, 0, H - 1).astype(jnp.int32)
    idx = yi * W + xi
    idx_ref[...] = idx[:, 0:64]
    bw_ref[...] = bw[:, 0:64]


def _prep(x, wv, bv, wsm, bsm, seg, wx, wy, bxy, axy):
    nblk = ROWS // _ROWBLK
    row_spec = lambda nc: pl.BlockSpec((_ROWBLK, nc), lambda i: (i, 0))
    full = lambda a: pl.BlockSpec(a.shape, lambda i: (0,) * a.ndim)
    return pl.pallas_call(
        _prep_body,
        grid=(nblk,),
        in_specs=[row_spec(C), full(wv), full(bv), full(wsm), full(bsm),
                  full(seg), full(wx), full(wy), full(bxy), row_spec(2)],
        out_specs=[row_spec(C), row_spec(64), row_spec(64), row_spec(128)],
        out_shape=[
            jax.ShapeDtypeStruct((ROWS, C), jnp.float32),
            jax.ShapeDtypeStruct((ROWS, 64), jnp.int32),
            jax.ShapeDtypeStruct((ROWS, 64), jnp.float32),
            jax.ShapeDtypeStruct((ROWS, 128), jnp.float32),
        ],
    )(x, wv, bv, wsm, bsm, seg, wx, wy, bxy, axy)


def _sc_body(v_hbm, idx_hbm, bw_hbm, sm_hbm, out_hbm,
             vslab, idx_v, bw_v, sm_v, out_v):
    wid = lax.axis_index("s") * 2 + lax.axis_index("c")
    iota16 = lax.iota(jnp.int32, LANE)
    for r in range(TASKS_PER_W):
        t = wid * TASKS_PER_W + r
        b = t // (NSLAB * 2)
        rem = t % (NSLAB * 2)
        slab = rem // 2
        half = rem % 2
        grp = slab // (NSLAB // G)
        row0 = b * HW + half * HALF
        col0 = slab * LANE
        pltpu.sync_copy(v_hbm.at[pl.ds(b * HW, HW), pl.ds(col0, LANE)],
                        vslab)
        pltpu.sync_copy(idx_hbm.at[pl.ds(row0, HALF)], idx_v)
        pltpu.sync_copy(bw_hbm.at[pl.ds(row0, HALF)], bw_v)
        pltpu.sync_copy(sm_hbm.at[pl.ds(row0, HALF), pl.ds(grp * LANE, LANE)],
                        sm_v)
        zeros16 = jnp.zeros((LANE,), jnp.int32)

        def loc(i, carry):
            smr = sm_v[i]
            idxr = [idx_v[i, pl.ds(c * LANE, LANE)] for c in range(NCORN)]
            bwr = [bw_v[i, pl.ds(c * LANE, LANE)] for c in range(NCORN)]
            acc = jnp.zeros((LANE,), jnp.float32)
            for p in range(P):
                smv = smr[p]
                ptmp = jnp.zeros((LANE,), jnp.float32)
                for c in range(NCORN):
                    rid = idxr[c][p]
                    bwv = bwr[c][p]
                    vals = vslab[rid]
                    ptmp = ptmp + vals * bwv
                acc = acc + ptmp * smv
            out_v[i] = acc
            return carry

        lax.fori_loop(0, HALF, loc, 0)
        pltpu.sync_copy(out_v,
                        out_hbm.at[pl.ds(row0, HALF), pl.ds(col0, LANE)])


@functools.lru_cache(maxsize=1)
def _sc_aggregate_fn():
    return pl.kernel(
        _sc_body,
        out_type=jax.ShapeDtypeStruct((ROWS, C), jnp.float32),
        mesh=plsc.VectorSubcoreMesh(core_axis_name="c",
                                    subcore_axis_name="s"),
        compiler_params=pltpu.CompilerParams(use_tc_tiling_on_sc=False,
                                             needs_layout_passes=False),
        scratch_types=[
            pltpu.VMEM((HW, LANE), jnp.float32),
            pltpu.VMEM((HALF, 64), jnp.int32),
            pltpu.VMEM((HALF, 64), jnp.float32),
            pltpu.VMEM((HALF, LANE), jnp.float32),
            pltpu.VMEM((HALF, LANE), jnp.float32),
        ],
    )


def _out_body(x_ref, w_ref, b_ref, o_ref):
    o_ref[...] = (jnp.dot(x_ref[...], w_ref[...],
                          preferred_element_type=jnp.float32) + b_ref[...])


def _out_proj(x, w_t, b):
    nblk = ROWS // _ROWBLK
    return pl.pallas_call(
        _out_body,
        grid=(nblk,),
        in_specs=[pl.BlockSpec((_ROWBLK, C), lambda i: (i, 0)),
                  pl.BlockSpec((C, C), lambda i: (0, 0)),
                  pl.BlockSpec((1, C), lambda i: (0, 0))],
        out_specs=pl.BlockSpec((_ROWBLK, C), lambda i: (i, 0)),
        out_shape=jax.ShapeDtypeStruct((ROWS, C), jnp.float32),
    )(x, w_t, b.reshape(1, C))


def _lane_maps():
    sm_cols = np.zeros((G * P, 128), np.float32)
    for g in range(G):
        for p in range(P):
            sm_cols[p * G + g, g * LANE + p] = 1.0
    seg = np.zeros((128, 128), np.float32)
    for g in range(G):
        for p in range(P):
            for q in range(P):
                seg[g * LANE + p, g * LANE + q] = 1.0
    offx = np.zeros((2 * P, 128), np.float32)
    offy = np.zeros((2 * P, 128), np.float32)
    for c in range(NCORN):
        for p in range(P):
            offx[p * 2 + 0, c * LANE + p] = float(W)
            offy[p * 2 + 1, c * LANE + p] = float(H)
    return sm_cols, seg, offx, offy


_SM_COLS, _SEG, _OFFX, _OFFY = _lane_maps()


@jax.jit
def kernel(feats, anchor_points, W_value, b_value, W_weights, b_weights,
           W_offset, b_offset, W_out, b_out):
    x = feats.reshape(BS, C, HW).transpose(0, 2, 1).reshape(ROWS, C)

    sm_cols = jnp.asarray(_SM_COLS)
    seg = jnp.asarray(_SEG)
    wsm = W_weights.T @ sm_cols
    bsm = (b_weights @ sm_cols).reshape(1, 128)
    wx = W_offset.T @ jnp.asarray(_OFFX)
    wy = W_offset.T @ jnp.asarray(_OFFY)
    bxy = jnp.stack([b_offset @ jnp.asarray(_OFFX),
                     b_offset @ jnp.asarray(_OFFY)])
    axy = (anchor_points.reshape(ROWS, 2) * jnp.float32(W)
           - jnp.float32(0.5))

    v, idx, bw, sm = _prep(x, W_value.T, b_value.reshape(1, C),
                           wsm, bsm, seg, wx, wy, bxy, axy)
    out_pre = _sc_aggregate_fn()(v, idx, bw, sm)
    out = _out_proj(out_pre, W_out.T, b_out)
    return out.reshape(BS, HW, C).transpose(0, 2, 1).reshape(BS, C, H, W)


# speedup vs baseline: 13.8864x; 1.1162x over previous
import functools

import jax
import jax.numpy as jnp
import numpy as np
from jax import lax
from jax.experimental import pallas as pl
from jax.experimental.pallas import tpu as pltpu
from jax.experimental.pallas import tpu_sc as plsc

C = 384
G = 8
P = 9
NCORN = 4
BS = 4
H = 32
W = 32
HW = H * W
ROWS = BS * HW
LANE = 16
NSLAB = C // LANE
HALF = HW // 2
NTASK = BS * NSLAB * 2
NWORK = 32
TASKS_PER_W = NTASK // NWORK

_ROWBLK = 512


def _prep_body(x_ref, wv_ref, bv_ref, wsm_ref, bsm_ref, seg_ref,
               wx_ref, wy_ref, bxy_ref, axy_ref,
               v_ref, idx_ref, bw_ref, sm_ref):
    x = x_ref[...]
    v_ref[...] = (jnp.dot(x, wv_ref[...], preferred_element_type=jnp.float32)
                  + bv_ref[...])
    e = jnp.exp(jnp.dot(x, wsm_ref[...], preferred_element_type=jnp.float32)
                + bsm_ref[...])
    z = jnp.dot(e, seg_ref[...], preferred_element_type=jnp.float32)
    sm_ref[...] = e / z
    xp = (jnp.dot(x, wx_ref[...], preferred_element_type=jnp.float32)
          + bxy_ref[0:1, :] + axy_ref[:, 0:1])
    yp = (jnp.dot(x, wy_ref[...], preferred_element_type=jnp.float32)
          + bxy_ref[1:2, :] + axy_ref[:, 1:2])
    lane = lax.broadcasted_iota(jnp.int32, xp.shape, 1)
    cidx = lane >> 4
    is_x1 = (cidx & 1) == 1
    is_y1 = cidx >= 2
    one = jnp.float32(1.0)
    xf = jnp.floor(xp)
    dx = xp - xf
    xc = xf + jnp.where(is_x1, one, 0.0)
    wxc = jnp.where(is_x1, dx, one - dx)
    yf = jnp.floor(yp)
    dy = yp - yf
    yc = yf + jnp.where(is_y1, one, 0.0)
    wyc = jnp.where(is_y1, dy, one - dy)
    valid = ((xc >= 0) & (xc <= W - 1) & (yc >= 0) & (yc <= H - 1))
    bw = wxc * wyc * jnp.where(valid, one, 0.0)
    xi = jnp.clip(xc, 0, W - 1).astype(jnp.int32)
    yi = jnp.clip(yc, 0, H - 1).astype(jnp.int32)
    idx = yi * W + xi
    idx_ref[...] = idx[:, 0:64]
    bw_ref[...] = bw[:, 0:64]


def _prep(x, wv, bv, wsm, bsm, seg, wx, wy, bxy, axy):
    nblk = ROWS // _ROWBLK
    row_spec = lambda nc: pl.BlockSpec((_ROWBLK, nc), lambda i: (i, 0))
    full = lambda a: pl.BlockSpec(a.shape, lambda i: (0,) * a.ndim)
    return pl.pallas_call(
        _prep_body,
        grid=(nblk,),
        in_specs=[row_spec(C), full(wv), full(bv), full(wsm), full(bsm),
                  full(seg), full(wx), full(wy), full(bxy), row_spec(2)],
        out_specs=[row_spec(C), row_spec(64), row_spec(64), row_spec(128)],
        out_shape=[
            jax.ShapeDtypeStruct((ROWS, C), jnp.float32),
            jax.ShapeDtypeStruct((ROWS, 64), jnp.int32),
            jax.ShapeDtypeStruct((ROWS, 64), jnp.float32),
            jax.ShapeDtypeStruct((ROWS, 128), jnp.float32),
        ],
    )(x, wv, bv, wsm, bsm, seg, wx, wy, bxy, axy)


def _sc_body(v_hbm, idx_hbm, bw_hbm, sm_hbm, out_hbm,
             vslab, idx_v, bw_v, sm_v, out_v):
    wid = lax.axis_index("s") * 2 + lax.axis_index("c")
    iota16 = lax.iota(jnp.int32, LANE)
    for r in range(TASKS_PER_W):
        t = wid * TASKS_PER_W + r
        b = t // (NSLAB * 2)
        rem = t % (NSLAB * 2)
        slab = rem // 2
        half = rem % 2
        grp = slab // (NSLAB // G)
        row0 = b * HW + half * HALF
        col0 = slab * LANE
        pltpu.sync_copy(v_hbm.at[pl.ds(b * HW, HW), pl.ds(col0, LANE)],
                        vslab)
        pltpu.sync_copy(idx_hbm.at[pl.ds(row0, HALF)], idx_v)
        pltpu.sync_copy(bw_hbm.at[pl.ds(row0, HALF)], bw_v)
        pltpu.sync_copy(sm_hbm.at[pl.ds(row0, HALF), pl.ds(grp * LANE, LANE)],
                        sm_v)
        zeros16 = jnp.zeros((LANE,), jnp.int32)

        @plsc.parallel_loop(0, HALF, unroll=2)
        def loc(i):
            smr = sm_v[i]
            idxr = [idx_v[i, pl.ds(c * LANE, LANE)] for c in range(NCORN)]
            bwr = [bw_v[i, pl.ds(c * LANE, LANE)] for c in range(NCORN)]
            acc = jnp.zeros((LANE,), jnp.float32)
            for p in range(P):
                smv = smr[p]
                ptmp = jnp.zeros((LANE,), jnp.float32)
                for c in range(NCORN):
                    rid = idxr[c][p]
                    bwv = bwr[c][p]
                    vals = vslab[rid]
                    ptmp = ptmp + vals * bwv
                acc = acc + ptmp * smv
            out_v[i] = acc
        pltpu.sync_copy(out_v,
                        out_hbm.at[pl.ds(row0, HALF), pl.ds(col0, LANE)])


@functools.lru_cache(maxsize=1)
def _sc_aggregate_fn():
    return pl.kernel(
        _sc_body,
        out_type=jax.ShapeDtypeStruct((ROWS, C), jnp.float32),
        mesh=plsc.VectorSubcoreMesh(core_axis_name="c",
                                    subcore_axis_name="s"),
        compiler_params=pltpu.CompilerParams(use_tc_tiling_on_sc=False,
                                             needs_layout_passes=False),
        scratch_types=[
            pltpu.VMEM((HW, LANE), jnp.float32),
            pltpu.VMEM((HALF, 64), jnp.int32),
            pltpu.VMEM((HALF, 64), jnp.float32),
            pltpu.VMEM((HALF, LANE), jnp.float32),
            pltpu.VMEM((HALF, LANE), jnp.float32),
        ],
    )


def _out_body(x_ref, w_ref, b_ref, o_ref):
    o_ref[...] = (jnp.dot(x_ref[...], w_ref[...],
                          preferred_element_type=jnp.float32) + b_ref[...])


def _out_proj(x, w_t, b):
    nblk = ROWS // _ROWBLK
    return pl.pallas_call(
        _out_body,
        grid=(nblk,),
        in_specs=[pl.BlockSpec((_ROWBLK, C), lambda i: (i, 0)),
                  pl.BlockSpec((C, C), lambda i: (0, 0)),
                  pl.BlockSpec((1, C), lambda i: (0, 0))],
        out_specs=pl.BlockSpec((_ROWBLK, C), lambda i: (i, 0)),
        out_shape=jax.ShapeDtypeStruct((ROWS, C), jnp.float32),
    )(x, w_t, b.reshape(1, C))


def _lane_maps():
    sm_cols = np.zeros((G * P, 128), np.float32)
    for g in range(G):
        for p in range(P):
            sm_cols[p * G + g, g * LANE + p] = 1.0
    seg = np.zeros((128, 128), np.float32)
    for g in range(G):
        for p in range(P):
            for q in range(P):
                seg[g * LANE + p, g * LANE + q] = 1.0
    offx = np.zeros((2 * P, 128), np.float32)
    offy = np.zeros((2 * P, 128), np.float32)
    for c in range(NCORN):
        for p in range(P):
            offx[p * 2 + 0, c * LANE + p] = float(W)
            offy[p * 2 + 1, c * LANE + p] = float(H)
    return sm_cols, seg, offx, offy


_SM_COLS, _SEG, _OFFX, _OFFY = _lane_maps()


@jax.jit
def kernel(feats, anchor_points, W_value, b_value, W_weights, b_weights,
           W_offset, b_offset, W_out, b_out):
    x = feats.reshape(BS, C, HW).transpose(0, 2, 1).reshape(ROWS, C)

    sm_cols = jnp.asarray(_SM_COLS)
    seg = jnp.asarray(_SEG)
    wsm = W_weights.T @ sm_cols
    bsm = (b_weights @ sm_cols).reshape(1, 128)
    wx = W_offset.T @ jnp.asarray(_OFFX)
    wy = W_offset.T @ jnp.asarray(_OFFY)
    bxy = jnp.stack([b_offset @ jnp.asarray(_OFFX),
                     b_offset @ jnp.asarray(_OFFY)])
    axy = (anchor_points.reshape(ROWS, 2) * jnp.float32(W)
           - jnp.float32(0.5))

    v, idx, bw, sm = _prep(x, W_value.T, b_value.reshape(1, C),
                           wsm, bsm, seg, wx, wy, bxy, axy)
    out_pre = _sc_aggregate_fn()(v, idx, bw, sm)
    out = _out_proj(out_pre, W_out.T, b_out)
    return out.reshape(BS, HW, C).transpose(0, 2, 1).reshape(BS, C, H, W)


# speedup vs baseline: 14.5058x; 1.0446x over previous
import functools

import jax
import jax.numpy as jnp
import numpy as np
from jax import lax
from jax.experimental import pallas as pl
from jax.experimental.pallas import tpu as pltpu
from jax.experimental.pallas import tpu_sc as plsc

C = 384
G = 8
P = 9
NCORN = 4
BS = 4
H = 32
W = 32
HW = H * W
ROWS = BS * HW
LANE = 16
NSLAB = C // LANE
HALF = HW // 2
NTASK = BS * NSLAB * 2
NWORK = 32
TASKS_PER_W = NTASK // NWORK

_ROWBLK = 512


def _prep_body(x_ref, wv_ref, bv_ref, wsm_ref, bsm_ref, seg_ref,
               wx_ref, wy_ref, bxy_ref, axy_ref,
               v_ref, idx_ref, bw_ref, sm_ref):
    x = x_ref[...]
    v_ref[...] = (jnp.dot(x, wv_ref[...], preferred_element_type=jnp.float32)
                  + bv_ref[...])
    e = jnp.exp(jnp.dot(x, wsm_ref[...], preferred_element_type=jnp.float32)
                + bsm_ref[...])
    z = jnp.dot(e, seg_ref[...], preferred_element_type=jnp.float32)
    sm_ref[...] = e / z
    xp = (jnp.dot(x, wx_ref[...], preferred_element_type=jnp.float32)
          + bxy_ref[0:1, :] + axy_ref[:, 0:1])
    yp = (jnp.dot(x, wy_ref[...], preferred_element_type=jnp.float32)
          + bxy_ref[1:2, :] + axy_ref[:, 1:2])
    lane = lax.broadcasted_iota(jnp.int32, xp.shape, 1)
    cidx = lane >> 4
    is_x1 = (cidx & 1) == 1
    is_y1 = cidx >= 2
    one = jnp.float32(1.0)
    xf = jnp.floor(xp)
    dx = xp - xf
    xc = xf + jnp.where(is_x1, one, 0.0)
    wxc = jnp.where(is_x1, dx, one - dx)
    yf = jnp.floor(yp)
    dy = yp - yf
    yc = yf + jnp.where(is_y1, one, 0.0)
    wyc = jnp.where(is_y1, dy, one - dy)
    valid = ((xc >= 0) & (xc <= W - 1) & (yc >= 0) & (yc <= H - 1))
    bw = wxc * wyc * jnp.where(valid, one, 0.0)
    xi = jnp.clip(xc, 0, W - 1).astype(jnp.int32)
    yi = jnp.clip(yc, 0, H - 1).astype(jnp.int32)
    idx = yi * W + xi
    idx_ref[...] = idx[:, 0:64]
    bw_ref[...] = bw[:, 0:64]


def _prep(x, wv, bv, wsm, bsm, seg, wx, wy, bxy, axy):
    nblk = ROWS // _ROWBLK
    row_spec = lambda nc: pl.BlockSpec((_ROWBLK, nc), lambda i: (i, 0))
    full = lambda a: pl.BlockSpec(a.shape, lambda i: (0,) * a.ndim)
    return pl.pallas_call(
        _prep_body,
        grid=(nblk,),
        in_specs=[row_spec(C), full(wv), full(bv), full(wsm), full(bsm),
                  full(seg), full(wx), full(wy), full(bxy), row_spec(2)],
        out_specs=[row_spec(C), row_spec(64), row_spec(64), row_spec(128)],
        out_shape=[
            jax.ShapeDtypeStruct((ROWS, C), jnp.float32),
            jax.ShapeDtypeStruct((ROWS, 64), jnp.int32),
            jax.ShapeDtypeStruct((ROWS, 64), jnp.float32),
            jax.ShapeDtypeStruct((ROWS, 128), jnp.float32),
        ],
    )(x, wv, bv, wsm, bsm, seg, wx, wy, bxy, axy)


def _sc_body(v_hbm, idx_hbm, bw_hbm, sm_hbm, out_hbm,
             vslab, idx_v, bw_v, sm_v, out_v):
    wid = lax.axis_index("s") * 2 + lax.axis_index("c")
    iota16 = lax.iota(jnp.int32, LANE)
    for r in range(TASKS_PER_W):
        t = wid * TASKS_PER_W + r
        b = t // (NSLAB * 2)
        rem = t % (NSLAB * 2)
        slab = rem // 2
        half = rem % 2
        grp = slab // (NSLAB // G)
        row0 = b * HW + half * HALF
        col0 = slab * LANE
        pltpu.sync_copy(v_hbm.at[pl.ds(b * HW, HW), pl.ds(col0, LANE)],
                        vslab)
        pltpu.sync_copy(idx_hbm.at[pl.ds(row0, HALF)], idx_v)
        pltpu.sync_copy(bw_hbm.at[pl.ds(row0, HALF)], bw_v)
        pltpu.sync_copy(sm_hbm.at[pl.ds(row0, HALF), pl.ds(grp * LANE, LANE)],
                        sm_v)
        zeros16 = jnp.zeros((LANE,), jnp.int32)

        @plsc.parallel_loop(0, HALF, unroll=4)
        def loc(i):
            smr = sm_v[i]
            idxr = [idx_v[i, pl.ds(c * LANE, LANE)] for c in range(NCORN)]
            bwr = [bw_v[i, pl.ds(c * LANE, LANE)] for c in range(NCORN)]
            acc = jnp.zeros((LANE,), jnp.float32)
            for p in range(P):
                smv = smr[p]
                ptmp = jnp.zeros((LANE,), jnp.float32)
                for c in range(NCORN):
                    rid = idxr[c][p]
                    bwv = bwr[c][p]
                    vals = vslab[rid]
                    ptmp = ptmp + vals * bwv
                acc = acc + ptmp * smv
            out_v[i] = acc
        pltpu.sync_copy(out_v,
                        out_hbm.at[pl.ds(row0, HALF), pl.ds(col0, LANE)])


@functools.lru_cache(maxsize=1)
def _sc_aggregate_fn():
    return pl.kernel(
        _sc_body,
        out_type=jax.ShapeDtypeStruct((ROWS, C), jnp.float32),
        mesh=plsc.VectorSubcoreMesh(core_axis_name="c",
                                    subcore_axis_name="s"),
        compiler_params=pltpu.CompilerParams(use_tc_tiling_on_sc=False,
                                             needs_layout_passes=False),
        scratch_types=[
            pltpu.VMEM((HW, LANE), jnp.float32),
            pltpu.VMEM((HALF, 64), jnp.int32),
            pltpu.VMEM((HALF, 64), jnp.float32),
            pltpu.VMEM((HALF, LANE), jnp.float32),
            pltpu.VMEM((HALF, LANE), jnp.float32),
        ],
    )


def _out_body(x_ref, w_ref, b_ref, o_ref):
    o_ref[...] = (jnp.dot(x_ref[...], w_ref[...],
                          preferred_element_type=jnp.float32) + b_ref[...])


def _out_proj(x, w_t, b):
    nblk = ROWS // _ROWBLK
    return pl.pallas_call(
        _out_body,
        grid=(nblk,),
        in_specs=[pl.BlockSpec((_ROWBLK, C), lambda i: (i, 0)),
                  pl.BlockSpec((C, C), lambda i: (0, 0)),
                  pl.BlockSpec((1, C), lambda i: (0, 0))],
        out_specs=pl.BlockSpec((_ROWBLK, C), lambda i: (i, 0)),
        out_shape=jax.ShapeDtypeStruct((ROWS, C), jnp.float32),
    )(x, w_t, b.reshape(1, C))


def _lane_maps():
    sm_cols = np.zeros((G * P, 128), np.float32)
    for g in range(G):
        for p in range(P):
            sm_cols[p * G + g, g * LANE + p] = 1.0
    seg = np.zeros((128, 128), np.float32)
    for g in range(G):
        for p in range(P):
            for q in range(P):
                seg[g * LANE + p, g * LANE + q] = 1.0
    offx = np.zeros((2 * P, 128), np.float32)
    offy = np.zeros((2 * P, 128), np.float32)
    for c in range(NCORN):
        for p in range(P):
            offx[p * 2 + 0, c * LANE + p] = float(W)
            offy[p * 2 + 1, c * LANE + p] = float(H)
    return sm_cols, seg, offx, offy


_SM_COLS, _SEG, _OFFX, _OFFY = _lane_maps()


@jax.jit
def kernel(feats, anchor_points, W_value, b_value, W_weights, b_weights,
           W_offset, b_offset, W_out, b_out):
    x = feats.reshape(BS, C, HW).transpose(0, 2, 1).reshape(ROWS, C)

    sm_cols = jnp.asarray(_SM_COLS)
    seg = jnp.asarray(_SEG)
    wsm = W_weights.T @ sm_cols
    bsm = (b_weights @ sm_cols).reshape(1, 128)
    wx = W_offset.T @ jnp.asarray(_OFFX)
    wy = W_offset.T @ jnp.asarray(_OFFY)
    bxy = jnp.stack([b_offset @ jnp.asarray(_OFFX),
                     b_offset @ jnp.asarray(_OFFY)])
    axy = (anchor_points.reshape(ROWS, 2) * jnp.float32(W)
           - jnp.float32(0.5))

    v, idx, bw, sm = _prep(x, W_value.T, b_value.reshape(1, C),
                           wsm, bsm, seg, wx, wy, bxy, axy)
    out_pre = _sc_aggregate_fn()(v, idx, bw, sm)
    out = _out_proj(out_pre, W_out.T, b_out)
    return out.reshape(BS, HW, C).transpose(0, 2, 1).reshape(BS, C, H, W)


# speedup vs baseline: 14.8096x; 1.0209x over previous
import functools

import jax
import jax.numpy as jnp
import numpy as np
from jax import lax
from jax.experimental import pallas as pl
from jax.experimental.pallas import tpu as pltpu
from jax.experimental.pallas import tpu_sc as plsc

C = 384
G = 8
P = 9
NCORN = 4
BS = 4
H = 32
W = 32
HW = H * W
ROWS = BS * HW
LANE = 16
NSLAB = C // LANE
HALF = HW // 2
NTASK = BS * NSLAB * 2
NWORK = 32
TASKS_PER_W = NTASK // NWORK

_ROWBLK = 512


_NT = (((1,), (1,)), ((), ()))


def _prep_body(x_ref, wv_ref, bv_ref, ww_ref, bw72_ref, sm_cols_ref, seg_ref,
               wo_ref, bo18_ref, offx_ref, offy_ref, axy_ref,
               v_ref, idx_ref, bw_ref, sm_ref):
    x = x_ref[...]
    f32 = jnp.float32
    v_ref[...] = (lax.dot_general(x, wv_ref[...], _NT,
                                  preferred_element_type=f32) + bv_ref[...])
    wr = (lax.dot_general(x, ww_ref[...], _NT, preferred_element_type=f32)
          + bw72_ref[...])
    e = jnp.exp(jnp.dot(wr, sm_cols_ref[...], preferred_element_type=f32))
    z = jnp.dot(e, seg_ref[...], preferred_element_type=f32)
    sm_ref[...] = e / z
    off = (lax.dot_general(x, wo_ref[...], _NT, preferred_element_type=f32)
           + bo18_ref[...])
    xp = (jnp.dot(off, offx_ref[...], preferred_element_type=f32)
          + axy_ref[:, 0:1])
    yp = (jnp.dot(off, offy_ref[...], preferred_element_type=f32)
          + axy_ref[:, 1:2])
    lane = lax.broadcasted_iota(jnp.int32, xp.shape, 1)
    cidx = lane >> 4
    is_x1 = (cidx & 1) == 1
    is_y1 = cidx >= 2
    one = jnp.float32(1.0)
    xf = jnp.floor(xp)
    dx = xp - xf
    xc = xf + jnp.where(is_x1, one, 0.0)
    wxc = jnp.where(is_x1, dx, one - dx)
    yf = jnp.floor(yp)
    dy = yp - yf
    yc = yf + jnp.where(is_y1, one, 0.0)
    wyc = jnp.where(is_y1, dy, one - dy)
    valid = ((xc >= 0) & (xc <= W - 1) & (yc >= 0) & (yc <= H - 1))
    bw = wxc * wyc * jnp.where(valid, one, 0.0)
    xi = jnp.clip(xc, 0, W - 1).astype(jnp.int32)
    yi = jnp.clip(yc, 0, H - 1).astype(jnp.int32)
    idx = yi * W + xi
    idx_ref[...] = idx[:, 0:64]
    bw_ref[...] = bw[:, 0:64]


def _prep(x, wv, bv, ww, bw72, sm_cols, seg, wo, bo18, offx, offy, axy):
    nblk = ROWS // _ROWBLK
    row_spec = lambda nc: pl.BlockSpec((_ROWBLK, nc), lambda i: (i, 0))
    full = lambda a: pl.BlockSpec(a.shape, lambda i: (0,) * a.ndim)
    return pl.pallas_call(
        _prep_body,
        grid=(nblk,),
        in_specs=[row_spec(C), full(wv), full(bv), full(ww), full(bw72),
                  full(sm_cols), full(seg), full(wo), full(bo18),
                  full(offx), full(offy), row_spec(2)],
        out_specs=[row_spec(C), row_spec(64), row_spec(64), row_spec(128)],
        out_shape=[
            jax.ShapeDtypeStruct((ROWS, C), jnp.float32),
            jax.ShapeDtypeStruct((ROWS, 64), jnp.int32),
            jax.ShapeDtypeStruct((ROWS, 64), jnp.float32),
            jax.ShapeDtypeStruct((ROWS, 128), jnp.float32),
        ],
    )(x, wv, bv, ww, bw72, sm_cols, seg, wo, bo18, offx, offy, axy)


def _sc_body(v_hbm, idx_hbm, bw_hbm, sm_hbm, out_hbm,
             vslab, idx_v, bw_v, sm_v, out_v):
    wid = lax.axis_index("s") * 2 + lax.axis_index("c")
    iota16 = lax.iota(jnp.int32, LANE)
    for r in range(TASKS_PER_W):
        t = wid * TASKS_PER_W + r
        b = t // (NSLAB * 2)
        rem = t % (NSLAB * 2)
        slab = rem // 2
        half = rem % 2
        grp = slab // (NSLAB // G)
        row0 = b * HW + half * HALF
        col0 = slab * LANE
        pltpu.sync_copy(v_hbm.at[pl.ds(b * HW, HW), pl.ds(col0, LANE)],
                        vslab)
        pltpu.sync_copy(idx_hbm.at[pl.ds(row0, HALF)], idx_v)
        pltpu.sync_copy(bw_hbm.at[pl.ds(row0, HALF)], bw_v)
        pltpu.sync_copy(sm_hbm.at[pl.ds(row0, HALF), pl.ds(grp * LANE, LANE)],
                        sm_v)
        zeros16 = jnp.zeros((LANE,), jnp.int32)

        @plsc.parallel_loop(0, HALF, unroll=4)
        def loc(i):
            smr = sm_v[i]
            idxr = [idx_v[i, pl.ds(c * LANE, LANE)] for c in range(NCORN)]
            bwr = [bw_v[i, pl.ds(c * LANE, LANE)] for c in range(NCORN)]
            acc = jnp.zeros((LANE,), jnp.float32)
            for p in range(P):
                smv = smr[p]
                ptmp = jnp.zeros((LANE,), jnp.float32)
                for c in range(NCORN):
                    rid = idxr[c][p]
                    bwv = bwr[c][p]
                    vals = vslab[rid]
                    ptmp = ptmp + vals * bwv
                acc = acc + ptmp * smv
            out_v[i] = acc
        pltpu.sync_copy(out_v,
                        out_hbm.at[pl.ds(row0, HALF), pl.ds(col0, LANE)])


@functools.lru_cache(maxsize=1)
def _sc_aggregate_fn():
    return pl.kernel(
        _sc_body,
        out_type=jax.ShapeDtypeStruct((ROWS, C), jnp.float32),
        mesh=plsc.VectorSubcoreMesh(core_axis_name="c",
                                    subcore_axis_name="s"),
        compiler_params=pltpu.CompilerParams(use_tc_tiling_on_sc=False,
                                             needs_layout_passes=False),
        scratch_types=[
            pltpu.VMEM((HW, LANE), jnp.float32),
            pltpu.VMEM((HALF, 64), jnp.int32),
            pltpu.VMEM((HALF, 64), jnp.float32),
            pltpu.VMEM((HALF, LANE), jnp.float32),
            pltpu.VMEM((HALF, LANE), jnp.float32),
        ],
    )


def _out_body(x_ref, w_ref, b_ref, o_ref):
    ot = (lax.dot_general(w_ref[...], x_ref[...], _NT,
                          preferred_element_type=jnp.float32)
          + b_ref[...])
    o_ref[...] = ot[None]


def _out_proj(x, w, b):
    nblk = ROWS // _ROWBLK
    per_b = HW // _ROWBLK
    return pl.pallas_call(
        _out_body,
        grid=(nblk,),
        in_specs=[pl.BlockSpec((_ROWBLK, C), lambda i: (i, 0)),
                  pl.BlockSpec((C, C), lambda i: (0, 0)),
                  pl.BlockSpec((C, 1), lambda i: (0, 0))],
        out_specs=pl.BlockSpec((1, C, _ROWBLK),
                               lambda i: (i // per_b, 0, i % per_b)),
        out_shape=jax.ShapeDtypeStruct((BS, C, HW), jnp.float32),
    )(x, w, b.reshape(C, 1))


def _lane_maps():
    sm_cols = np.zeros((G * P, 128), np.float32)
    for g in range(G):
        for p in range(P):
            sm_cols[p * G + g, g * LANE + p] = 1.0
    seg = np.zeros((128, 128), np.float32)
    for g in range(G):
        for p in range(P):
            for q in range(P):
                seg[g * LANE + p, g * LANE + q] = 1.0
    offx = np.zeros((2 * P, 128), np.float32)
    offy = np.zeros((2 * P, 128), np.float32)
    for c in range(NCORN):
        for p in range(P):
            offx[p * 2 + 0, c * LANE + p] = float(W)
            offy[p * 2 + 1, c * LANE + p] = float(H)
    return sm_cols, seg, offx, offy


_SM_COLS, _SEG, _OFFX, _OFFY = _lane_maps()


@jax.jit
def kernel(feats, anchor_points, W_value, b_value, W_weights, b_weights,
           W_offset, b_offset, W_out, b_out):
    x = feats.reshape(BS, C, HW).transpose(0, 2, 1).reshape(ROWS, C)

    axy = (anchor_points.reshape(ROWS, 2) * jnp.float32(W)
           - jnp.float32(0.5))

    v, idx, bw, sm = _prep(x, W_value, b_value.reshape(1, C),
                           W_weights, b_weights.reshape(1, G * P),
                           jnp.asarray(_SM_COLS), jnp.asarray(_SEG),
                           W_offset, b_offset.reshape(1, 2 * P),
                           jnp.asarray(_OFFX), jnp.asarray(_OFFY), axy)
    out_pre = _sc_aggregate_fn()(v, idx, bw, sm)
    out = _out_proj(out_pre, W_out, b_out)
    return out.reshape(BS, C, H, W)
